# baseline (speedup 1.0000x reference)
import functools

import jax
import jax.numpy as jnp
from jax import lax
from jax.experimental import pallas as pl
from jax.experimental.pallas import tpu as pltpu
from jax.experimental.pallas import tpu_sc as plsc

N = 10000
E = 320000
D = 128
H = 128
ALPHA = 0.01

NC = 2
NS = 16
HH = H // NC
K = 128
NCH = 160
EPT = NCH * K
EPAD = NS * EPT
LANES = 16
RPT = 624
TAIL = N - RPT * NS


def _mm_body(d_ref, n_ref, w_ref, o_ref):
    feat = d_ref[...] + ALPHA * n_ref[...]
    y = jnp.dot(feat, w_ref[...], preferred_element_type=jnp.float32)
    o_ref[0] = y[:, :HH]
    o_ref[1] = y[:, HH:]


def _matmul(data, noise, W):
    blk = 1000
    return pl.pallas_call(
        _mm_body,
        grid=(N // blk,),
        in_specs=[
            pl.BlockSpec((blk, D), lambda i: (i, 0)),
            pl.BlockSpec((blk, D), lambda i: (i, 0)),
            pl.BlockSpec((D, H), lambda i: (0, 0)),
        ],
        out_specs=pl.BlockSpec((NC, blk, HH), lambda i: (0, i, 0)),
        out_shape=jax.ShapeDtypeStruct((NC, N, HH), jnp.float32),
    )(data, noise, W)


NBUF = 4


def _spmm_body(xs_hbm, rc_hbm, val_hbm, out_hbm,
               rc_v, val_v, bufs, cols, rowb, acc, gsems):
    cid = lax.axis_index("c")
    sid = lax.axis_index("s")
    x_hbm = xs_hbm.at[cid]
    buf0 = bufs[0]

    pltpu.sync_copy(rc_hbm.at[sid], rc_v)
    pltpu.sync_copy(val_hbm.at[sid], val_v)

    def _unpack_col(j, cb):
        for g in range(K // LANES):
            sl = pl.ds(g * LANES, LANES)
            cb[sl] = jnp.bitwise_and(rc_v[j, sl], 0xFFFF)

    def _unpack_row(j):
        for g in range(K // LANES):
            sl = pl.ds(g * LANES, LANES)
            rowb[sl] = jnp.right_shift(rc_v[j, sl], 16)

    zeros16 = jnp.zeros((LANES,), jnp.float32)

    def _zrow(e, carry):
        for u in range(HH // LANES):
            buf0[e, pl.ds(u * LANES, LANES)] = zeros16
        return carry

    lax.fori_loop(0, K, _zrow, 0)

    base_r = pl.multiple_of(sid * RPT, 8)
    rem = RPT % K
    for q in range(RPT // K):
        pltpu.sync_copy(buf0, acc.at[pl.ds(base_r + q * K, K)])
    if rem:
        pltpu.sync_copy(buf0.at[pl.ds(0, rem)],
                        acc.at[pl.ds(base_r + (RPT // K) * K, rem)])

    @pl.when(sid == NS - 1)
    def _zero_tail():
        pltpu.sync_copy(buf0.at[pl.ds(0, TAIL)], acc.at[pl.ds(RPT * NS, TAIL)])

    plsc.subcore_barrier()

    def _scale_scatter(j, buf):
        for g in range(K // LANES):
            vv = val_v[j, pl.ds(g * LANES, LANES)]
            for e16 in range(LANES):
                v = vv[e16]
                e = g * LANES + e16
                for u in range(HH // LANES):
                    sl = pl.ds(u * LANES, LANES)
                    buf[e, sl] = buf[e, sl] * v
        _unpack_row(j)
        pltpu.sync_copy(buf, acc.at[rowb], add=True)

    dummy = x_hbm.at[pl.ds(0, K)]
    for p in range(NBUF):
        _unpack_col(p, cols[p])
        pltpu.async_copy(x_hbm.at[cols[p]], bufs[p], gsems[p])

    def _ring(t, carry):
        for p in range(NBUF):
            j = NBUF * t + p
            pltpu.make_async_copy(dummy, bufs[p], gsems[p]).wait()
            _scale_scatter(j, bufs[p])
            jn = jnp.minimum(j + NBUF, NCH - 1)
            _unpack_col(jn, cols[p])
            pltpu.async_copy(x_hbm.at[cols[p]], bufs[p], gsems[p])
        return carry

    lax.fori_loop(0, NCH // NBUF, _ring, 0)
    for p in range(NBUF):
        pltpu.make_async_copy(dummy, bufs[p], gsems[p]).wait()
    plsc.subcore_barrier()

    for q in range(RPT // K):
        sl = pl.ds(base_r + q * K, K)
        pltpu.sync_copy(acc.at[sl], out_hbm.at[cid].at[sl])
    if rem:
        sl = pl.ds(base_r + (RPT // K) * K, rem)
        pltpu.sync_copy(acc.at[sl], out_hbm.at[cid].at[sl])

    @pl.when(sid == NS - 1)
    def _drain_tail():
        sl = pl.ds(RPT * NS, TAIL)
        pltpu.sync_copy(acc.at[sl], out_hbm.at[cid].at[sl])


@functools.cache
def _make_spmm():
    return pl.kernel(
        _spmm_body,
        out_type=jax.ShapeDtypeStruct((NC, N, HH), jnp.float32),
        mesh=plsc.VectorSubcoreMesh(core_axis_name="c", subcore_axis_name="s",
                                    num_cores=NC, num_subcores=NS),
        compiler_params=pltpu.CompilerParams(use_tc_tiling_on_sc=False),
        scratch_types=[
            pltpu.VMEM((NCH, K), jnp.int32),
            pltpu.VMEM((NCH, K), jnp.float32),
            [pltpu.VMEM((K, HH), jnp.float32) for _ in range(NBUF)],
            [pltpu.VMEM((K,), jnp.int32) for _ in range(NBUF)],
            pltpu.VMEM((K,), jnp.int32),
            pltpu.VMEM_SHARED((N, HH), jnp.float32),
            [pltpu.SemaphoreType.DMA for _ in range(NBUF)],
        ],
    )


def _elu_body(p_ref, o_ref):
    for c in range(NC):
        s = p_ref[c]
        o_ref[:, c * HH:(c + 1) * HH] = jnp.where(
            s > 0, s, jnp.exp(jnp.minimum(s, 0.0)) - 1.0)


def _elu_concat(partials):
    blk = 1000
    return pl.pallas_call(
        _elu_body,
        grid=(N // blk,),
        in_specs=[pl.BlockSpec((NC, blk, HH), lambda i: (0, i, 0))],
        out_specs=pl.BlockSpec((blk, H), lambda i: (i, 0)),
        out_shape=jax.ShapeDtypeStruct((N, H), jnp.float32),
    )(partials)


def kernel(data, adj_indices, adj_values, W):
    noise = jax.random.normal(jax.random.key(42), (N, D), dtype=jnp.float32)
    xs = _matmul(data, noise, W)
    pad = EPAD - E
    rc = jnp.left_shift(adj_indices[0], 16) | adj_indices[1]
    rc = jnp.pad(rc, (0, pad)).reshape(NS, NCH, K)
    val = jnp.pad(adj_values, (0, pad)).reshape(NS, NCH, K)
    partials = _make_spmm()(xs, rc, val)
    return _elu_concat(partials)


# speedup vs baseline: 1.6760x; 1.6760x over previous
import functools

import jax
import jax.numpy as jnp
from jax import lax
from jax.experimental import pallas as pl
from jax.experimental.pallas import tpu as pltpu
from jax.experimental.pallas import tpu_sc as plsc

N = 10000
E = 320000
D = 128
H = 128
ALPHA = 0.01

NC = 2
NS = 16
HH = H // NC
K = 128
NCH = 160
EPT = NCH * K
EPAD = NS * EPT
LANES = 16
RPT = 624
TAIL = N - RPT * NS


def _mm_body(d_ref, n_ref, w_ref, o_ref):
    feat = d_ref[...] + ALPHA * n_ref[...]
    y = jnp.dot(feat, w_ref[...], preferred_element_type=jnp.float32)
    o_ref[0] = y[:, :HH]
    o_ref[1] = y[:, HH:]


def _matmul(data, noise, W):
    blk = 1000
    return pl.pallas_call(
        _mm_body,
        grid=(N // blk,),
        in_specs=[
            pl.BlockSpec((blk, D), lambda i: (i, 0)),
            pl.BlockSpec((blk, D), lambda i: (i, 0)),
            pl.BlockSpec((D, H), lambda i: (0, 0)),
        ],
        out_specs=pl.BlockSpec((NC, blk, HH), lambda i: (0, i, 0)),
        out_shape=jax.ShapeDtypeStruct((NC, N, HH), jnp.float32),
    )(data, noise, W)


NBUF = 2


def _spmm_body(xs_hbm, rc_hbm, val_hbm, out_hbm,
               rc_v, vbufs, bufs, cols, rowb, xsp, acc, gsems, vsems):
    cid = lax.axis_index("c")
    sid = lax.axis_index("s")
    x_hbm = xs_hbm.at[cid]
    buf0 = bufs[0]

    pltpu.sync_copy(rc_hbm.at[sid], rc_v)

    def _unpack_col(j, cb):
        for g in range(K // LANES):
            sl = pl.ds(g * LANES, LANES)
            cb[sl] = jnp.bitwise_and(rc_v[j, sl], 0xFFFF)

    def _unpack_row(j):
        for g in range(K // LANES):
            sl = pl.ds(g * LANES, LANES)
            rowb[sl] = jnp.right_shift(rc_v[j, sl], 16)

    zeros16 = jnp.zeros((LANES,), jnp.float32)

    def _zrow(e, carry):
        for u in range(HH // LANES):
            buf0[e, pl.ds(u * LANES, LANES)] = zeros16
        return carry

    lax.fori_loop(0, K, _zrow, 0)

    base_r = pl.multiple_of(sid * RPT, 8)
    rem = RPT % K
    for q in range(RPT // K):
        pltpu.sync_copy(buf0, acc.at[pl.ds(base_r + q * K, K)])
    if rem:
        pltpu.sync_copy(buf0.at[pl.ds(0, rem)],
                        acc.at[pl.ds(base_r + (RPT // K) * K, rem)])

    @pl.when(sid == NS - 1)
    def _zero_tail():
        pltpu.sync_copy(buf0.at[pl.ds(0, TAIL)], acc.at[pl.ds(RPT * NS, TAIL)])

    pltpu.sync_copy(x_hbm.at[pl.ds(base_r, RPT)], xsp.at[pl.ds(base_r, RPT)])

    @pl.when(sid == NS - 1)
    def _stage_tail():
        sl = pl.ds(RPT * NS, TAIL)
        pltpu.sync_copy(x_hbm.at[sl], xsp.at[sl])

    plsc.subcore_barrier()

    def _scale_scatter(j, buf, vbuf):
        for g in range(K // LANES):
            vv = vbuf[pl.ds(g * LANES, LANES)]
            for e16 in range(LANES):
                v = vv[e16]
                e = g * LANES + e16
                for u in range(HH // LANES):
                    sl = pl.ds(u * LANES, LANES)
                    buf[e, sl] = buf[e, sl] * v
        _unpack_row(j)
        pltpu.sync_copy(buf, acc.at[rowb], add=True)

    dummy = x_hbm.at[pl.ds(0, K)]
    vdummy = val_hbm.at[0, 0]
    for p in range(NBUF):
        _unpack_col(p, cols[p])
        pltpu.async_copy(xsp.at[cols[p]], bufs[p], gsems[p])
        pltpu.async_copy(val_hbm.at[sid, p], vbufs[p], vsems[p])

    def _ring(t, carry):
        for p in range(NBUF):
            j = NBUF * t + p
            pltpu.make_async_copy(dummy, bufs[p], gsems[p]).wait()
            pltpu.make_async_copy(vdummy, vbufs[p], vsems[p]).wait()
            _scale_scatter(j, bufs[p], vbufs[p])
            jn = jnp.minimum(j + NBUF, NCH - 1)
            _unpack_col(jn, cols[p])
            pltpu.async_copy(xsp.at[cols[p]], bufs[p], gsems[p])
            pltpu.async_copy(val_hbm.at[sid, jn], vbufs[p], vsems[p])
        return carry

    lax.fori_loop(0, NCH // NBUF, _ring, 0)
    for p in range(NBUF):
        pltpu.make_async_copy(dummy, bufs[p], gsems[p]).wait()
        pltpu.make_async_copy(vdummy, vbufs[p], vsems[p]).wait()
    plsc.subcore_barrier()

    for q in range(RPT // K):
        sl = pl.ds(base_r + q * K, K)
        pltpu.sync_copy(acc.at[sl], out_hbm.at[cid].at[sl])
    if rem:
        sl = pl.ds(base_r + (RPT // K) * K, rem)
        pltpu.sync_copy(acc.at[sl], out_hbm.at[cid].at[sl])

    @pl.when(sid == NS - 1)
    def _drain_tail():
        sl = pl.ds(RPT * NS, TAIL)
        pltpu.sync_copy(acc.at[sl], out_hbm.at[cid].at[sl])


@functools.cache
def _make_spmm():
    return pl.kernel(
        _spmm_body,
        out_type=jax.ShapeDtypeStruct((NC, N, HH), jnp.float32),
        mesh=plsc.VectorSubcoreMesh(core_axis_name="c", subcore_axis_name="s",
                                    num_cores=NC, num_subcores=NS),
        compiler_params=pltpu.CompilerParams(use_tc_tiling_on_sc=False),
        scratch_types=[
            pltpu.VMEM((NCH, K), jnp.int32),
            [pltpu.VMEM((K,), jnp.float32) for _ in range(NBUF)],
            [pltpu.VMEM((K, HH), jnp.float32) for _ in range(NBUF)],
            [pltpu.VMEM((K,), jnp.int32) for _ in range(NBUF)],
            pltpu.VMEM((K,), jnp.int32),
            pltpu.VMEM_SHARED((N, HH), jnp.float32),
            pltpu.VMEM_SHARED((N, HH), jnp.float32),
            [pltpu.SemaphoreType.DMA for _ in range(NBUF)],
            [pltpu.SemaphoreType.DMA for _ in range(NBUF)],
        ],
    )


def _elu_body(p_ref, o_ref):
    for c in range(NC):
        s = p_ref[c]
        o_ref[:, c * HH:(c + 1) * HH] = jnp.where(
            s > 0, s, jnp.exp(jnp.minimum(s, 0.0)) - 1.0)


def _elu_concat(partials):
    blk = 1000
    return pl.pallas_call(
        _elu_body,
        grid=(N // blk,),
        in_specs=[pl.BlockSpec((NC, blk, HH), lambda i: (0, i, 0))],
        out_specs=pl.BlockSpec((blk, H), lambda i: (i, 0)),
        out_shape=jax.ShapeDtypeStruct((N, H), jnp.float32),
    )(partials)


def kernel(data, adj_indices, adj_values, W):
    noise = jax.random.normal(jax.random.key(42), (N, D), dtype=jnp.float32)
    xs = _matmul(data, noise, W)
    pad = EPAD - E
    rc = jnp.left_shift(adj_indices[0], 16) | adj_indices[1]
    rc = jnp.pad(rc, (0, pad)).reshape(NS, NCH, K)
    val = jnp.pad(adj_values, (0, pad)).reshape(NS, NCH, K)
    partials = _make_spmm()(xs, rc, val)
    return _elu_concat(partials)
